# baseline (speedup 1.0000x reference)
import functools

import jax
import jax.numpy as jnp
from jax import lax
from jax.experimental import pallas as pl
from jax.experimental.pallas import tpu as pltpu
from jax.experimental.pallas import tpu_sc as plsc

HID = 64
VOCAB = 64
SLOTS = 8
SEQ = 32
LANES = 16
NW = 32


def _prep_body(embT_ref, sW1T_ref, sb1_ref, sW2T_ref, sb2_ref, rW1T_ref,
               ts_ref, wcatT_ref):
    embT = embT_ref[:, 0:VOCAB]
    hT = jnp.maximum(jnp.dot(sW1T_ref[...], embT) + sb1_ref[...], 0.0)
    ts_ref[...] = jnp.dot(sW2T_ref[...], hT) + sb2_ref[...]
    wcatT_ref[:, 0:VOCAB] = jnp.dot(rW1T_ref[:, 0:HID], embT)
    wcatT_ref[:, VOCAB:2 * VOCAB] = (
        jnp.dot(rW1T_ref[:, HID:2 * HID], embT) * (1.0 / SLOTS))


def _prep(embed, sW1, sb1, sW2, sb2, rW1):
    return pl.pallas_call(
        _prep_body,
        out_shape=(
            jax.ShapeDtypeStruct((1, VOCAB), jnp.float32),
            jax.ShapeDtypeStruct((HID, 2 * VOCAB), jnp.float32),
        ),
    )(embed.T, sW1.T, sb1.reshape(-1, 1), sW2.T, sb2.reshape(-1, 1), rW1.T)


def _sim_body(seq4_hbm, ts_hbm, cnt4_hbm, ts_v, seq_v, cnt_v):
    wid = lax.axis_index("s") * 2 + lax.axis_index("c")
    n_cb = seq4_hbm.shape[1]
    cb_w = n_cb // NW
    per_w = cb_w * 128
    groups = per_w // LANES

    pltpu.sync_copy(ts_hbm, ts_v)
    pltpu.sync_copy(seq4_hbm.at[:, pl.ds(wid * cb_w, cb_w), :, :], seq_v)

    zeros_f = jnp.zeros((LANES,), jnp.float32)
    ones_f = jnp.full((LANES,), 1.0, jnp.float32)
    lane = lax.iota(jnp.int32, LANES)

    def fill(g):
        jg = jnp.full((LANES,), 0, jnp.int32) + g // 8
        kg = lane + (g % 8) * 16

        for rb in range(VOCAB // 8):
            for rr in range(8):
                cnt_v[rb, g // 8, rr, pl.ds((g % 8) * LANES, LANES)] = zeros_f

        tok = []
        sc = []
        for t in range(SLOTS):
            tt = plsc.load_gather(
                seq_v, [jnp.full((LANES,), t // 8, jnp.int32), jg,
                        jnp.full((LANES,), t % 8, jnp.int32), kg])
            tok.append(tt)
            sc.append(plsc.load_gather(ts_v, [tt]))
        return jg, kg, tok + sc

    def evict(t, jg, kg, st):
        tok = list(st[:SLOTS])
        sc = list(st[SLOTS:])
        ntok = plsc.load_gather(
            seq_v, [jnp.zeros((LANES,), jnp.int32) + t // 8, jg,
                    jnp.zeros((LANES,), jnp.int32) + t % 8, kg])
        nsc = plsc.load_gather(ts_v, [ntok])
        p1 = jnp.minimum(sc[0], sc[1])
        m23 = jnp.minimum(sc[2], sc[3])
        m45 = jnp.minimum(sc[4], sc[5])
        m67 = jnp.minimum(sc[6], sc[7])
        p2 = jnp.minimum(p1, sc[2])
        p3 = jnp.minimum(p1, m23)
        m47 = jnp.minimum(m45, m67)
        p4 = jnp.minimum(p3, sc[4])
        p5 = jnp.minimum(p3, m45)
        p6 = jnp.minimum(p5, sc[6])
        m = jnp.minimum(p3, m47)
        pref = [sc[0], p1, p2, p3, p4, p5, p6]
        ntoks = []
        nscs = []
        for i in range(SLOTS - 1):
            keep = pref[i] > m
            ntoks.append(jnp.where(keep, tok[i], tok[i + 1]))
            nscs.append(jnp.where(keep, sc[i], sc[i + 1]))
        return tuple(ntoks + [ntok] + nscs + [nsc])

    def group_pair(gg, carry):
        jg0, kg0, st0 = fill(gg * 2)
        jg1, kg1, st1 = fill(gg * 2 + 1)

        def step(t, c):
            return (evict(t, jg0, kg0, c[0]), evict(t, jg1, kg1, c[1]))

        fin0, fin1 = lax.fori_loop(SLOTS, SEQ - 1, step, (tuple(st0), tuple(st1)))
        for fin, jg, kg in ((fin0, jg0, kg0), (fin1, jg1, kg1)):
            for i in range(SLOTS):
                plsc.addupdate_scatter(
                    cnt_v, [fin[i] >> 3, jg, fin[i] & 7, kg], ones_f)
        return carry

    lax.fori_loop(0, groups // 2, group_pair, 0)
    pltpu.sync_copy(cnt_v, cnt4_hbm.at[:, pl.ds(wid * cb_w, cb_w), :, :])


def _sim(seq4, ts):
    n_cb = seq4.shape[1]
    mesh = plsc.VectorSubcoreMesh(core_axis_name="c", subcore_axis_name="s")
    cb_w = n_cb // NW
    f = functools.partial(
        pl.kernel,
        out_type=jax.ShapeDtypeStruct((VOCAB // 8, n_cb, 8, 128), jnp.float32),
        mesh=mesh,
        scratch_types=[
            pltpu.VMEM((VOCAB,), jnp.float32),
            pltpu.VMEM((SEQ // 8, cb_w, 8, 128), jnp.int32),
            pltpu.VMEM((VOCAB // 8, cb_w, 8, 128), jnp.float32),
        ],
        compiler_params=pltpu.CompilerParams(
            needs_layout_passes=False, use_tc_tiling_on_sc=False),
    )(_sim_body)
    return f(seq4, ts)


def _head_body(cntT_ref, q_ref, wcatT_ref, rb1_ref, rW2T_ref, rb2_ref, out_ref):
    blk = cntT_ref.shape[1]
    iota = lax.broadcasted_iota(jnp.int32, (VOCAB, blk), 0)
    qohT = (q_ref[...] == iota).astype(jnp.float32)
    hT = (jnp.dot(wcatT_ref[:, 0:VOCAB], qohT)
          + jnp.dot(wcatT_ref[:, VOCAB:2 * VOCAB], cntT_ref[...])
          + rb1_ref[...])
    hT = jnp.maximum(hT, 0.0)
    out_ref[...] = jnp.dot(rW2T_ref[...], hT) + rb2_ref[...]


def _head(cntT, query_tok, wcatT, rb1, rW2, rb2):
    B = cntT.shape[1]
    blk = 4096
    outT = pl.pallas_call(
        _head_body,
        grid=(B // blk,),
        in_specs=[
            pl.BlockSpec((VOCAB, blk), lambda i: (0, i)),
            pl.BlockSpec((1, blk), lambda i: (0, i)),
            pl.BlockSpec((HID, 2 * VOCAB), lambda i: (0, 0)),
            pl.BlockSpec((HID, 1), lambda i: (0, 0)),
            pl.BlockSpec((HID, VOCAB), lambda i: (0, 0)),
            pl.BlockSpec((VOCAB, 1), lambda i: (0, 0)),
        ],
        out_specs=pl.BlockSpec((VOCAB, blk), lambda i: (0, i)),
        out_shape=jax.ShapeDtypeStruct((VOCAB, B), jnp.float32),
        compiler_params=pltpu.CompilerParams(
            dimension_semantics=("parallel",)),
    )(cntT, query_tok.reshape(1, B), wcatT, rb1.reshape(-1, 1),
      rW2.T, rb2.reshape(-1, 1))
    return outT.T


def kernel(seqs, query_tok, embed, sW1, sb1, sW2, sb2, rW1, rb1, rW2, rb2):
    B = seqs.shape[0]
    seqs = seqs.astype(jnp.int32)
    query_tok = query_tok.astype(jnp.int32)
    tsT, wcatT = _prep(embed, sW1, sb1, sW2, sb2, rW1)
    seq4 = seqs.T.reshape(SEQ // 8, 8, B // 128, 128).transpose(0, 2, 1, 3)
    cnt4 = _sim(seq4, tsT.reshape(VOCAB))
    cntT = cnt4.transpose(0, 2, 1, 3).reshape(VOCAB, B)
    return _head(cntT, query_tok, wcatT, rb1, rW2, rb2)


# speedup vs baseline: 1.0171x; 1.0171x over previous
import functools

import jax
import jax.numpy as jnp
from jax import lax
from jax.experimental import pallas as pl
from jax.experimental.pallas import tpu as pltpu
from jax.experimental.pallas import tpu_sc as plsc

HID = 64
VOCAB = 64
SLOTS = 8
SEQ = 32
LANES = 16
NW = 32


def _prep_body(embT_ref, sW1T_ref, sb1_ref, sW2T_ref, sb2_ref, rW1T_ref,
               ts_ref, wcatT_ref):
    embT = embT_ref[:, 0:VOCAB]
    hT = jnp.maximum(jnp.dot(sW1T_ref[...], embT) + sb1_ref[...], 0.0)
    ts_ref[...] = jnp.dot(sW2T_ref[...], hT) + sb2_ref[...]
    wcatT_ref[:, 0:VOCAB] = jnp.dot(rW1T_ref[:, 0:HID], embT)
    wcatT_ref[:, VOCAB:2 * VOCAB] = (
        jnp.dot(rW1T_ref[:, HID:2 * HID], embT) * (1.0 / SLOTS))


def _prep(embed, sW1, sb1, sW2, sb2, rW1):
    return pl.pallas_call(
        _prep_body,
        out_shape=(
            jax.ShapeDtypeStruct((1, VOCAB), jnp.float32),
            jax.ShapeDtypeStruct((HID, 2 * VOCAB), jnp.float32),
        ),
    )(embed.T, sW1.T, sb1.reshape(-1, 1), sW2.T, sb2.reshape(-1, 1), rW1.T)


def _sim_body(seq4_hbm, ts_hbm, cnt4_hbm, ts_v, seq_v, cnt_v):
    wid = lax.axis_index("s") * 2 + lax.axis_index("c")
    n_cb = seq4_hbm.shape[1]
    cb_w = n_cb // NW
    per_w = cb_w * 128
    groups = per_w // LANES

    pltpu.sync_copy(ts_hbm, ts_v)
    pltpu.sync_copy(seq4_hbm.at[:, pl.ds(wid * cb_w, cb_w), :, :], seq_v)

    zeros_f = jnp.zeros((LANES,), jnp.float32)
    ones_f = jnp.full((LANES,), 1.0, jnp.float32)
    lane = lax.iota(jnp.int32, LANES)

    def fill(g):
        jg = jnp.full((LANES,), 0, jnp.int32) + g // 8
        kg = lane + (g % 8) * 16

        for rb in range(VOCAB // 8):
            for rr in range(8):
                cnt_v[rb, g // 8, rr, pl.ds((g % 8) * LANES, LANES)] = zeros_f

        tok = []
        sc = []
        for t in range(SLOTS):
            tt = plsc.load_gather(
                seq_v, [jnp.full((LANES,), t // 8, jnp.int32), jg,
                        jnp.full((LANES,), t % 8, jnp.int32), kg])
            tok.append(tt)
            sc.append(plsc.load_gather(ts_v, [tt]))
        return jg, kg, tok + sc

    def evict(t, jg, kg, st):
        tok = list(st[:SLOTS])
        sc = list(st[SLOTS:])
        ntok = plsc.load_gather(
            seq_v, [jnp.zeros((LANES,), jnp.int32) + t // 8, jg,
                    jnp.zeros((LANES,), jnp.int32) + t % 8, kg])
        nsc = plsc.load_gather(ts_v, [ntok])
        pref = [sc[0]]
        for i in range(1, SLOTS - 1):
            pref.append(jnp.minimum(pref[-1], sc[i]))
        m = jnp.minimum(pref[-1], sc[SLOTS - 1])
        ntoks = []
        nscs = []
        for i in range(SLOTS - 1):
            keep = pref[i] > m
            ntoks.append(jnp.where(keep, tok[i], tok[i + 1]))
            nscs.append(jnp.where(keep, sc[i], sc[i + 1]))
        return tuple(ntoks + [ntok] + nscs + [nsc])

    def group_pair(gg, carry):
        jg0, kg0, st0 = fill(gg * 2)
        jg1, kg1, st1 = fill(gg * 2 + 1)

        def step(t, c):
            return (evict(t, jg0, kg0, c[0]), evict(t, jg1, kg1, c[1]))

        fin0, fin1 = lax.fori_loop(SLOTS, SEQ - 1, step, (tuple(st0), tuple(st1)))
        for fin, jg, kg in ((fin0, jg0, kg0), (fin1, jg1, kg1)):
            for i in range(SLOTS):
                plsc.addupdate_scatter(
                    cnt_v, [fin[i] >> 3, jg, fin[i] & 7, kg], ones_f)
        return carry

    lax.fori_loop(0, groups // 2, group_pair, 0)
    pltpu.sync_copy(cnt_v, cnt4_hbm.at[:, pl.ds(wid * cb_w, cb_w), :, :])


def _sim(seq4, ts):
    n_cb = seq4.shape[1]
    mesh = plsc.VectorSubcoreMesh(core_axis_name="c", subcore_axis_name="s")
    cb_w = n_cb // NW
    f = functools.partial(
        pl.kernel,
        out_type=jax.ShapeDtypeStruct((VOCAB // 8, n_cb, 8, 128), jnp.float32),
        mesh=mesh,
        scratch_types=[
            pltpu.VMEM((VOCAB,), jnp.float32),
            pltpu.VMEM((SEQ // 8, cb_w, 8, 128), jnp.int32),
            pltpu.VMEM((VOCAB // 8, cb_w, 8, 128), jnp.float32),
        ],
        compiler_params=pltpu.CompilerParams(
            needs_layout_passes=False, use_tc_tiling_on_sc=False),
    )(_sim_body)
    return f(seq4, ts)


def _head_body(cntT_ref, q_ref, wcatT_ref, rb1_ref, rW2T_ref, rb2_ref, out_ref):
    blk = cntT_ref.shape[1]
    iota = lax.broadcasted_iota(jnp.int32, (VOCAB, blk), 0)
    qohT = (q_ref[...] == iota).astype(jnp.float32)
    hT = (jnp.dot(wcatT_ref[:, 0:VOCAB], qohT)
          + jnp.dot(wcatT_ref[:, VOCAB:2 * VOCAB], cntT_ref[...])
          + rb1_ref[...])
    hT = jnp.maximum(hT, 0.0)
    out_ref[...] = jnp.dot(rW2T_ref[...], hT) + rb2_ref[...]


def _head(cntT, query_tok, wcatT, rb1, rW2, rb2):
    B = cntT.shape[1]
    blk = 4096
    outT = pl.pallas_call(
        _head_body,
        grid=(B // blk,),
        in_specs=[
            pl.BlockSpec((VOCAB, blk), lambda i: (0, i)),
            pl.BlockSpec((1, blk), lambda i: (0, i)),
            pl.BlockSpec((HID, 2 * VOCAB), lambda i: (0, 0)),
            pl.BlockSpec((HID, 1), lambda i: (0, 0)),
            pl.BlockSpec((HID, VOCAB), lambda i: (0, 0)),
            pl.BlockSpec((VOCAB, 1), lambda i: (0, 0)),
        ],
        out_specs=pl.BlockSpec((VOCAB, blk), lambda i: (0, i)),
        out_shape=jax.ShapeDtypeStruct((VOCAB, B), jnp.float32),
        compiler_params=pltpu.CompilerParams(
            dimension_semantics=("parallel",)),
    )(cntT, query_tok.reshape(1, B), wcatT, rb1.reshape(-1, 1),
      rW2.T, rb2.reshape(-1, 1))
    return outT.T


def kernel(seqs, query_tok, embed, sW1, sb1, sW2, sb2, rW1, rb1, rW2, rb2):
    B = seqs.shape[0]
    seqs = seqs.astype(jnp.int32)
    query_tok = query_tok.astype(jnp.int32)
    tsT, wcatT = _prep(embed, sW1, sb1, sW2, sb2, rW1)
    seq4 = seqs.T.reshape(SEQ // 8, 8, B // 128, 128).transpose(0, 2, 1, 3)
    cnt4 = _sim(seq4, tsT.reshape(VOCAB))
    cntT = cnt4.transpose(0, 2, 1, 3).reshape(VOCAB, B)
    return _head(cntT, query_tok, wcatT, rb1, rW2, rb2)


# speedup vs baseline: 1.0393x; 1.0218x over previous
import functools

import jax
import jax.numpy as jnp
from jax import lax
from jax.experimental import pallas as pl
from jax.experimental.pallas import tpu as pltpu
from jax.experimental.pallas import tpu_sc as plsc

HID = 64
VOCAB = 64
SLOTS = 8
SEQ = 32
LANES = 16
NW = 32


def _prep_body(embT_ref, sW1T_ref, sb1_ref, sW2T_ref, sb2_ref, rW1T_ref,
               ts_ref, wcatT_ref):
    embT = embT_ref[:, 0:VOCAB]
    hT = jnp.maximum(jnp.dot(sW1T_ref[...], embT) + sb1_ref[...], 0.0)
    ts_ref[...] = jnp.dot(sW2T_ref[...], hT) + sb2_ref[...]
    wcatT_ref[:, 0:VOCAB] = jnp.dot(rW1T_ref[:, 0:HID], embT)
    wcatT_ref[:, VOCAB:2 * VOCAB] = (
        jnp.dot(rW1T_ref[:, HID:2 * HID], embT) * (1.0 / SLOTS))


def _prep(embed, sW1, sb1, sW2, sb2, rW1):
    return pl.pallas_call(
        _prep_body,
        out_shape=(
            jax.ShapeDtypeStruct((1, VOCAB), jnp.float32),
            jax.ShapeDtypeStruct((HID, 2 * VOCAB), jnp.float32),
        ),
    )(embed.T, sW1.T, sb1.reshape(-1, 1), sW2.T, sb2.reshape(-1, 1), rW1.T)


def _sim_body(seq4_hbm, ts_hbm, cnt4_hbm, ts_v, seq_v, cnt_v, sem0, sem1):
    wid = lax.axis_index("s") * 2 + lax.axis_index("c")
    n_cb = seq4_hbm.shape[1]
    cb_w = n_cb // NW
    per_w = cb_w * 128
    groups = per_w // LANES

    ts_cp = pltpu.async_copy(ts_hbm, ts_v, sem0)
    seq_cp = pltpu.async_copy(
        seq4_hbm.at[:, pl.ds(wid * cb_w, cb_w), :, :], seq_v, sem1)
    ts_cp.wait()
    seq_cp.wait()

    zeros_f = jnp.zeros((LANES,), jnp.float32)
    ones_f = jnp.full((LANES,), 1.0, jnp.float32)
    lane = lax.iota(jnp.int32, LANES)

    def fill(g):
        jg = jnp.full((LANES,), 0, jnp.int32) + g // 8
        kg = lane + (g % 8) * 16

        for rb in range(VOCAB // 8):
            for rr in range(8):
                cnt_v[rb, g // 8, rr, pl.ds((g % 8) * LANES, LANES)] = zeros_f

        tok = []
        sc = []
        for t in range(SLOTS):
            tt = plsc.load_gather(
                seq_v, [jnp.full((LANES,), t // 8, jnp.int32), jg,
                        jnp.full((LANES,), t % 8, jnp.int32), kg])
            tok.append(tt)
            sc.append(plsc.load_gather(ts_v, [tt]))
        return jg, kg, tok + sc

    def evict(t, jg, kg, st):
        tok = list(st[:SLOTS])
        sc = list(st[SLOTS:])
        ntok = plsc.load_gather(
            seq_v, [jnp.zeros((LANES,), jnp.int32) + t // 8, jg,
                    jnp.zeros((LANES,), jnp.int32) + t % 8, kg])
        nsc = plsc.load_gather(ts_v, [ntok])
        pref = [sc[0]]
        for i in range(1, SLOTS - 1):
            pref.append(jnp.minimum(pref[-1], sc[i]))
        m = jnp.minimum(pref[-1], sc[SLOTS - 1])
        ntoks = []
        nscs = []
        for i in range(SLOTS - 1):
            keep = pref[i] > m
            ntoks.append(jnp.where(keep, tok[i], tok[i + 1]))
            nscs.append(jnp.where(keep, sc[i], sc[i + 1]))
        return tuple(ntoks + [ntok] + nscs + [nsc])

    def group_pair(gg, carry):
        jg0, kg0, st0 = fill(gg * 2)
        jg1, kg1, st1 = fill(gg * 2 + 1)

        def step(t, c):
            return (evict(t, jg0, kg0, c[0]), evict(t, jg1, kg1, c[1]))

        fin0, fin1 = lax.fori_loop(SLOTS, SEQ - 1, step, (tuple(st0), tuple(st1)))
        for fin, jg, kg in ((fin0, jg0, kg0), (fin1, jg1, kg1)):
            for i in range(SLOTS):
                plsc.addupdate_scatter(
                    cnt_v, [fin[i] >> 3, jg, fin[i] & 7, kg], ones_f)
        return carry

    lax.fori_loop(0, groups // 2, group_pair, 0)
    pltpu.sync_copy(cnt_v, cnt4_hbm.at[:, pl.ds(wid * cb_w, cb_w), :, :])


def _sim(seq4, ts):
    n_cb = seq4.shape[1]
    mesh = plsc.VectorSubcoreMesh(core_axis_name="c", subcore_axis_name="s")
    cb_w = n_cb // NW
    f = functools.partial(
        pl.kernel,
        out_type=jax.ShapeDtypeStruct((VOCAB // 8, n_cb, 8, 128), jnp.float32),
        mesh=mesh,
        scratch_types=[
            pltpu.VMEM((VOCAB,), jnp.float32),
            pltpu.VMEM((SEQ // 8, cb_w, 8, 128), jnp.int32),
            pltpu.VMEM((VOCAB // 8, cb_w, 8, 128), jnp.float32),
            pltpu.SemaphoreType.DMA,
            pltpu.SemaphoreType.DMA,
        ],
        compiler_params=pltpu.CompilerParams(
            needs_layout_passes=False, use_tc_tiling_on_sc=False),
    )(_sim_body)
    return f(seq4, ts)


def _head_body(cntT_ref, q_ref, wcatT_ref, rb1_ref, rW2T_ref, rb2_ref, out_ref):
    blk = cntT_ref.shape[1]
    iota = lax.broadcasted_iota(jnp.int32, (VOCAB, blk), 0)
    qohT = (q_ref[...] == iota).astype(jnp.float32)
    hT = (jnp.dot(wcatT_ref[:, 0:VOCAB], qohT)
          + jnp.dot(wcatT_ref[:, VOCAB:2 * VOCAB], cntT_ref[...])
          + rb1_ref[...])
    hT = jnp.maximum(hT, 0.0)
    out_ref[...] = jnp.dot(rW2T_ref[...], hT) + rb2_ref[...]


def _head(cntT, query_tok, wcatT, rb1, rW2, rb2):
    B = cntT.shape[1]
    blk = 4096
    outT = pl.pallas_call(
        _head_body,
        grid=(B // blk,),
        in_specs=[
            pl.BlockSpec((VOCAB, blk), lambda i: (0, i)),
            pl.BlockSpec((1, blk), lambda i: (0, i)),
            pl.BlockSpec((HID, 2 * VOCAB), lambda i: (0, 0)),
            pl.BlockSpec((HID, 1), lambda i: (0, 0)),
            pl.BlockSpec((HID, VOCAB), lambda i: (0, 0)),
            pl.BlockSpec((VOCAB, 1), lambda i: (0, 0)),
        ],
        out_specs=pl.BlockSpec((VOCAB, blk), lambda i: (0, i)),
        out_shape=jax.ShapeDtypeStruct((VOCAB, B), jnp.float32),
        compiler_params=pltpu.CompilerParams(
            dimension_semantics=("parallel",)),
    )(cntT, query_tok.reshape(1, B), wcatT, rb1.reshape(-1, 1),
      rW2.T, rb2.reshape(-1, 1))
    return outT.T


def kernel(seqs, query_tok, embed, sW1, sb1, sW2, sb2, rW1, rb1, rW2, rb2):
    B = seqs.shape[0]
    seqs = seqs.astype(jnp.int32)
    query_tok = query_tok.astype(jnp.int32)
    tsT, wcatT = _prep(embed, sW1, sb1, sW2, sb2, rW1)
    seq4 = seqs.T.reshape(SEQ // 8, 8, B // 128, 128).transpose(0, 2, 1, 3)
    cnt4 = _sim(seq4, tsT.reshape(VOCAB))
    cntT = cnt4.transpose(0, 2, 1, 3).reshape(VOCAB, B)
    return _head(cntT, query_tok, wcatT, rb1, rW2, rb2)
